# Initial kernel scaffold; baseline (speedup 1.0000x reference)
import jax
import jax.numpy as jnp
from jax.experimental import pallas as pl


def kernel(x, W1, b1, g1, be1, W2, b2, g2, be2, W3, b3, edge_index):
    raise NotImplementedError("write your pallas kernel here")


# speedup vs baseline: 13.0892x; 13.0892x over previous
import functools

import jax
import jax.numpy as jnp
from jax import lax
from jax.experimental import pallas as pl
from jax.experimental.pallas import tpu as pltpu
from jax.experimental.pallas import tpu_sc as plsc

_N = 10000
_NPAD = 10240
_E = 320000
_NC, _NS = 2, 16
_CH = 128
_NCH = 79
_EPT = _NCH * _CH
_EPAD = _EPT * _NC * _NS
_STRIPE = _NPAD // _NS
_DH = 128
_DP3 = 48


def _sc_mesh():
    return plsc.VectorSubcoreMesh(
        core_axis_name="c", subcore_axis_name="s",
        num_cores=_NC, num_subcores=_NS)


def _make_deg():
    @functools.partial(
        pl.kernel,
        out_type=jax.ShapeDtypeStruct((_NC, _NPAD), jnp.float32),
        mesh=_sc_mesh(),
        scratch_types=[
            pltpu.VMEM((_NCH, _CH), jnp.int32),
            pltpu.VMEM((_STRIPE,), jnp.float32),
            pltpu.VMEM((_CH,), jnp.float32),
            pltpu.VMEM_SHARED((_NPAD,), jnp.float32),
        ],
    )
    def deg_kernel(dst_hbm, out_hbm, dst_v, zbuf, ones_v, acc):
        c = lax.axis_index("c")
        s = lax.axis_index("s")
        zero16 = jnp.zeros((16,), jnp.float32)
        one16 = jnp.ones((16,), jnp.float32)
        for i in range(_STRIPE // 16):
            zbuf[pl.ds(i * 16, 16)] = zero16
        for i in range(_CH // 16):
            ones_v[pl.ds(i * 16, 16)] = one16
        pltpu.sync_copy(zbuf, acc.at[pl.ds(s * _STRIPE, _STRIPE)])
        plsc.subcore_barrier()
        pltpu.sync_copy(dst_hbm.at[c, s], dst_v)

        def body(j, carry):
            pltpu.sync_copy(ones_v, acc.at[dst_v.at[j]], add=True)
            return carry

        lax.fori_loop(0, _NCH, body, 0)
        plsc.subcore_barrier()
        pltpu.sync_copy(acc.at[pl.ds(s * _STRIPE, _STRIPE)],
                        out_hbm.at[c, pl.ds(s * _STRIPE, _STRIPE)])

    return deg_kernel


def _make_agg(d):
    @functools.partial(
        pl.kernel,
        out_type=jax.ShapeDtypeStruct((_NC, _NPAD, d), jnp.float32),
        mesh=_sc_mesh(),
        scratch_types=[
            pltpu.VMEM((_NCH, _CH), jnp.int32),
            pltpu.VMEM((_NCH, _CH), jnp.int32),
            pltpu.VMEM((_CH, d), jnp.float32),
            pltpu.VMEM_SHARED((_NPAD, d), jnp.float32),
        ],
        compiler_params=pltpu.CompilerParams(use_tc_tiling_on_sc=False),
    )
    def agg_kernel(hs_hbm, src_hbm, dst_hbm, out_hbm, src_v, dst_v, rows_v, acc):
        c = lax.axis_index("c")
        s = lax.axis_index("s")
        zero16 = jnp.zeros((16,), jnp.float32)

        def zrow(r, carry):
            for k in range(d // 16):
                rows_v[r, pl.ds(k * 16, 16)] = zero16
            return carry

        lax.fori_loop(0, _CH, zrow, 0)
        for k in range(_STRIPE // _CH):
            pltpu.sync_copy(rows_v,
                            acc.at[pl.ds(s * _STRIPE + k * _CH, _CH)])
        plsc.subcore_barrier()
        pltpu.sync_copy(src_hbm.at[c, s], src_v)
        pltpu.sync_copy(dst_hbm.at[c, s], dst_v)

        def body(j, carry):
            pltpu.sync_copy(hs_hbm.at[src_v.at[j]], rows_v)
            pltpu.sync_copy(rows_v, acc.at[dst_v.at[j]], add=True)
            return carry

        lax.fori_loop(0, _NCH, body, 0)
        plsc.subcore_barrier()
        pltpu.sync_copy(acc.at[pl.ds(s * _STRIPE, _STRIPE)],
                        out_hbm.at[c, pl.ds(s * _STRIPE, _STRIPE)])

    return agg_kernel


def _tc_a(x_ref, w_ref, degt_ref, hs_ref, dinv_ref):
    deg = jnp.sum(degt_ref[...], axis=1, keepdims=True) + 1.0
    dinv = lax.rsqrt(deg)
    h = jnp.dot(x_ref[...], w_ref[...], preferred_element_type=jnp.float32)
    hs_ref[...] = h * dinv
    dinv_ref[...] = dinv


def _tc_b(p_ref, hs_ref, dinv_ref, b_ref, g_ref, be_ref, w_ref, out_ref):
    dinv = dinv_ref[...]
    t = (p_ref[0] + p_ref[1] + hs_ref[...]) * dinv + b_ref[...]
    rows = lax.broadcasted_iota(jnp.int32, t.shape, 0)
    t = jnp.where(rows < _N, t, 0.0)
    m = jnp.sum(t, axis=0, keepdims=True) * (1.0 / _N)
    v = jnp.sum(t * t, axis=0, keepdims=True) * (1.0 / _N) - m * m
    a = (t - m) * lax.rsqrt(v + 1e-5) * g_ref[...] + be_ref[...]
    a = jnp.maximum(a, 0.0)
    a = jnp.where(rows < _N, a, 0.0)
    h = jnp.dot(a, w_ref[...], preferred_element_type=jnp.float32)
    out_ref[...] = h * dinv


def _tc_c(p_ref, hs_ref, dinv_ref, b_ref, out_ref):
    t = (p_ref[0] + p_ref[1] + hs_ref[...]) * dinv_ref[...]
    t = t[:_N, :40] + b_ref[...]
    mx = jnp.max(t, axis=1, keepdims=True)
    lse = jnp.log(jnp.sum(jnp.exp(t - mx), axis=1, keepdims=True)) + mx
    out_ref[...] = t - lse


def kernel(x, W1, b1, g1, be1, W2, b2, g2, be2, W3, b3, edge_index):
    fill = jnp.full((_EPAD - _E,), _N, jnp.int32)
    src4 = jnp.concatenate([edge_index[0], fill]).reshape(_NC, _NS, _NCH, _CH)
    dst4 = jnp.concatenate([edge_index[1], fill]).reshape(_NC, _NS, _NCH, _CH)
    x_pad = jnp.pad(x, ((0, _NPAD - _N), (0, 0)))
    w3p = jnp.pad(W3, ((0, 0), (0, _DP3 - 40)))

    degp = _make_deg()(dst4)
    degt = degp.T

    f32 = jnp.float32
    hs1, dinv = pl.pallas_call(
        _tc_a,
        out_shape=[jax.ShapeDtypeStruct((_NPAD, _DH), f32),
                   jax.ShapeDtypeStruct((_NPAD, 1), f32)],
    )(x_pad, W1, degt)

    agg128 = _make_agg(_DH)
    p1 = agg128(hs1, src4, dst4)
    hs2 = pl.pallas_call(
        _tc_b, out_shape=jax.ShapeDtypeStruct((_NPAD, _DH), f32),
    )(p1, hs1, dinv, b1, g1, be1, W2)

    p2 = agg128(hs2, src4, dst4)
    hs3 = pl.pallas_call(
        _tc_b, out_shape=jax.ShapeDtypeStruct((_NPAD, _DP3), f32),
    )(p2, hs2, dinv, b2, g2, be2, w3p)

    p3 = _make_agg(_DP3)(hs3, src4, dst4)
    out = pl.pallas_call(
        _tc_c, out_shape=jax.ShapeDtypeStruct((_N, 40), f32),
    )(p3, hs3, dinv, b3)
    return out
